# Initial kernel scaffold; baseline (speedup 1.0000x reference)
import jax
import jax.numpy as jnp
from jax.experimental import pallas as pl


def kernel(x, edge_index, W1l, b1l, W1r, W2l, b2l, W2r, W3l, b3l, W3r, Wfc, bfc):
    raise NotImplementedError("write your pallas kernel here")


# speedup vs baseline: 8.6867x; 8.6867x over previous
import functools

import jax
import jax.numpy as jnp
from jax import lax
from jax.experimental import pallas as pl
from jax.experimental.pallas import tpu as pltpu
from jax.experimental.pallas import tpu_sc as plsc

N = 10000
E = 320000
D = 128
NPAD = 10240

NC = 2
NS = 16
NW = NC * NS
EPW = E // NW
CH = 125
NCHUNK = EPW // CH
PCH = 40
BUF = 128
RPT = NPAD // NS

_MESH = plsc.VectorSubcoreMesh(
    core_axis_name="c", subcore_axis_name="s", num_cores=NC, num_subcores=NS
)


def _zero_vmem_rows(ref, nrows, ncols):
    zvec = jnp.zeros((16,), jnp.float32)

    def zrow(i, carry):
        for j in range(ncols // 16):
            ref[i, pl.ds(j * 16, 16)] = zvec
        return carry

    lax.fori_loop(0, nrows, zrow, 0)


def _sc_agg_body(src2_hbm, dst2_hbm, h_hbm, acc_out, sidx_v, didx_v,
                 buf0, buf1, acc_s, gsem0, gsem1, ssem0, ssem1):
    c = lax.axis_index("c")
    s = lax.axis_index("s")
    wid = s * NC + c
    r0 = s * RPT
    bufs = (buf0, buf1)
    gsems = (gsem0, gsem1)
    ssems = (ssem0, ssem1)

    _zero_vmem_rows(buf0, BUF, D)
    for j in range(RPT // BUF):
        pltpu.sync_copy(buf0, acc_s.at[pl.ds(r0 + j * BUF, BUF)])
    plsc.subcore_barrier()

    for ph in range(NCHUNK // PCH):
        base_row = wid * NCHUNK + ph * PCH
        pltpu.sync_copy(src2_hbm.at[pl.ds(base_row, PCH)], sidx_v)
        pltpu.sync_copy(dst2_hbm.at[pl.ds(base_row, PCH)], didx_v)
        pltpu.async_copy(h_hbm.at[sidx_v.at[0]], buf0.at[pl.ds(0, CH)], gsem0)

        def step(g, carry):
            for b in range(2):
                i = g * 2 + b
                nb = 1 - b

                pltpu.make_async_copy(h_hbm.at[sidx_v.at[i]],
                                      bufs[b].at[pl.ds(0, CH)], gsems[b]).wait()
                pltpu.async_copy(bufs[b].at[pl.ds(0, CH)],
                                 acc_s.at[didx_v.at[i]], ssems[b], add=True)

                @pl.when(i >= 1)
                def _():
                    pltpu.make_async_copy(bufs[nb].at[pl.ds(0, CH)],
                                          acc_s.at[didx_v.at[i - 1]],
                                          ssems[nb]).wait()

                @pl.when(i + 1 < PCH)
                def _():
                    pltpu.async_copy(h_hbm.at[sidx_v.at[i + 1]],
                                     bufs[nb].at[pl.ds(0, CH)], gsems[nb])
            return carry

        lax.fori_loop(0, PCH // 2, step, 0)
        pltpu.make_async_copy(bufs[1].at[pl.ds(0, CH)],
                              acc_s.at[didx_v.at[PCH - 1]], ssems[1]).wait()
    plsc.subcore_barrier()

    for j in range(RPT // BUF):
        pltpu.sync_copy(acc_s.at[pl.ds(r0 + j * BUF, BUF)], buf0)
        pltpu.sync_copy(buf0, acc_out.at[c, pl.ds(r0 + j * BUF, BUF)])


_sc_agg = pl.kernel(
    _sc_agg_body,
    out_type=[jax.ShapeDtypeStruct((NC, NPAD, D), jnp.float32)],
    mesh=_MESH,
    scratch_types=[
        pltpu.VMEM((PCH, CH), jnp.int32),
        pltpu.VMEM((PCH, CH), jnp.int32),
        pltpu.VMEM((BUF, D), jnp.float32),
        pltpu.VMEM((BUF, D), jnp.float32),
        pltpu.VMEM_SHARED((NPAD, D), jnp.float32),
        pltpu.SemaphoreType.DMA,
        pltpu.SemaphoreType.DMA,
        pltpu.SemaphoreType.DMA,
        pltpu.SemaphoreType.DMA,
    ],
)


def _sc_deg_body(dst2_hbm, deg_out, didx_v, buf, acc_s):
    c = lax.axis_index("c")
    s = lax.axis_index("s")
    wid = s * NC + c
    r0 = s * RPT

    pltpu.sync_copy(dst2_hbm.at[pl.ds(wid * NCHUNK, NCHUNK)], didx_v)
    _zero_vmem_rows(buf, BUF, D)
    for j in range(RPT // BUF):
        pltpu.sync_copy(buf, acc_s.at[pl.ds(r0 + j * BUF, BUF)])
    plsc.subcore_barrier()

    onevec = jnp.full((16,), 1.0, jnp.float32)

    def orow(i, carry):
        for j in range(D // 16):
            buf[i, pl.ds(j * 16, 16)] = onevec
        return carry

    lax.fori_loop(0, BUF, orow, 0)

    def body(g, carry):
        pltpu.sync_copy(buf.at[pl.ds(0, CH)], acc_s.at[didx_v.at[g]], add=True)
        return carry

    lax.fori_loop(0, NCHUNK, body, 0)
    plsc.subcore_barrier()

    for j in range(RPT // BUF):
        pltpu.sync_copy(acc_s.at[pl.ds(r0 + j * BUF, BUF)], buf)
        pltpu.sync_copy(buf, deg_out.at[c, pl.ds(r0 + j * BUF, BUF)])


_sc_deg = pl.kernel(
    _sc_deg_body,
    out_type=[jax.ShapeDtypeStruct((NC, NPAD, D), jnp.float32)],
    mesh=_MESH,
    scratch_types=[
        pltpu.VMEM((NCHUNK, CH), jnp.int32),
        pltpu.VMEM((BUF, D), jnp.float32),
        pltpu.VMEM_SHARED((NPAD, D), jnp.float32),
    ],
)


_RB = 400


def _layer1_body(acc, deg, h, Wl, bl, Wr, out, inv_out):
    d = deg[0, :, 0:1] + deg[1, :, 0:1]
    inv = 1.0 / jnp.maximum(d, 1.0)
    inv_out[...] = inv
    agg = (acc[0] + acc[1]) * inv
    y = (jnp.dot(agg, Wl[...], preferred_element_type=jnp.float32)
         + jnp.dot(h[...], Wr[...], preferred_element_type=jnp.float32)
         + bl[...])
    out[...] = jnp.maximum(y, 0.0)


def _tc_layer1(acc, deg, h, Wl, bl, Wr):
    return pl.pallas_call(
        _layer1_body,
        grid=(N // _RB,),
        in_specs=[
            pl.BlockSpec((NC, _RB, D), lambda i: (0, i, 0)),
            pl.BlockSpec((NC, _RB, D), lambda i: (0, i, 0)),
            pl.BlockSpec((_RB, D), lambda i: (i, 0)),
            pl.BlockSpec((D, D), lambda i: (0, 0)),
            pl.BlockSpec((1, D), lambda i: (0, 0)),
            pl.BlockSpec((D, D), lambda i: (0, 0)),
        ],
        out_specs=[
            pl.BlockSpec((_RB, D), lambda i: (i, 0)),
            pl.BlockSpec((_RB, 1), lambda i: (i, 0)),
        ],
        out_shape=[
            jax.ShapeDtypeStruct((N, D), jnp.float32),
            jax.ShapeDtypeStruct((N, 1), jnp.float32),
        ],
    )(acc, deg, h, Wl, bl.reshape(1, D), Wr)


def _layer_body(relu, fc, *refs):
    if fc:
        acc, inv, h, Wl, bl, Wr, Wfc, bfc, out = refs
    else:
        acc, inv, h, Wl, bl, Wr, out = refs
    agg = (acc[0] + acc[1]) * inv[...]
    y = (jnp.dot(agg, Wl[...], preferred_element_type=jnp.float32)
         + jnp.dot(h[...], Wr[...], preferred_element_type=jnp.float32)
         + bl[...])
    if relu:
        y = jnp.maximum(y, 0.0)
    if fc:
        y = jnp.dot(y, Wfc[...], preferred_element_type=jnp.float32) + bfc[...]
    out[...] = y


def _tc_layer(acc, inv, h, Wl, bl, Wr, Wfc=None, bfc=None, relu=True):
    fc = Wfc is not None
    in_specs = [
        pl.BlockSpec((NC, _RB, D), lambda i: (0, i, 0)),
        pl.BlockSpec((_RB, 1), lambda i: (i, 0)),
        pl.BlockSpec((_RB, D), lambda i: (i, 0)),
        pl.BlockSpec((D, D), lambda i: (0, 0)),
        pl.BlockSpec((1, D), lambda i: (0, 0)),
        pl.BlockSpec((D, D), lambda i: (0, 0)),
    ]
    args = [acc, inv, h, Wl, bl.reshape(1, D), Wr]
    if fc:
        in_specs += [
            pl.BlockSpec((D, D), lambda i: (0, 0)),
            pl.BlockSpec((1, D), lambda i: (0, 0)),
        ]
        args += [Wfc, bfc]
    return pl.pallas_call(
        functools.partial(_layer_body, relu, fc),
        grid=(N // _RB,),
        in_specs=in_specs,
        out_specs=pl.BlockSpec((_RB, D), lambda i: (i, 0)),
        out_shape=jax.ShapeDtypeStruct((N, D), jnp.float32),
    )(*args)


def kernel(x, edge_index, W1l, b1l, W1r, W2l, b2l, W2r, W3l, b3l, W3r, Wfc, bfc):
    src = edge_index[0].reshape(NW * NCHUNK, CH)
    dst = edge_index[1].reshape(NW * NCHUNK, CH)

    (acc1,) = _sc_agg(src, dst, x)
    (deg,) = _sc_deg(dst)
    h1, inv = _tc_layer1(acc1, deg, x, W1l, b1l, W1r)
    (acc2,) = _sc_agg(src, dst, h1)
    h2 = _tc_layer(acc2, inv, h1, W2l, b2l, W2r)
    (acc3,) = _sc_agg(src, dst, h2)
    Wfc_p = jnp.zeros((D, D), jnp.float32).at[:, : Wfc.shape[1]].set(Wfc)
    bfc_p = jnp.zeros((1, D), jnp.float32).at[0, : bfc.shape[0]].set(bfc)
    pre = _tc_layer(acc3, inv, h2, W3l, b3l, W3r, Wfc_p, bfc_p, relu=False)
    return pre[:, : Wfc.shape[1]]


# speedup vs baseline: 9.6928x; 1.1158x over previous
import functools

import jax
import jax.numpy as jnp
from jax import lax
from jax.experimental import pallas as pl
from jax.experimental.pallas import tpu as pltpu
from jax.experimental.pallas import tpu_sc as plsc

N = 10000
E = 320000
D = 128
NPAD = 10240

NC = 2
NS = 16
NW = NC * NS
EPW = E // NW
CH = 125
NCHUNK = EPW // CH
PCH = 40
BUF = 128
RPT = NPAD // NS

_MESH = plsc.VectorSubcoreMesh(
    core_axis_name="c", subcore_axis_name="s", num_cores=NC, num_subcores=NS
)


def _zero_vmem_rows(ref, nrows, ncols):
    zvec = jnp.zeros((16,), jnp.float32)

    def zrow(i, carry):
        for j in range(ncols // 16):
            ref[i, pl.ds(j * 16, 16)] = zvec
        return carry

    lax.fori_loop(0, nrows, zrow, 0)


def _sc_agg_body(src2_hbm, dst2_hbm, h_hbm, acc_out, sidx_v, didx_v,
                 buf0, buf1, acc_s, gsem0, gsem1):
    c = lax.axis_index("c")
    s = lax.axis_index("s")
    wid = s * NC + c
    r0 = s * RPT
    bufs = (buf0, buf1)
    gsems = (gsem0, gsem1)

    _zero_vmem_rows(buf0, BUF, D)
    for j in range(RPT // BUF):
        pltpu.sync_copy(buf0, acc_s.at[pl.ds(r0 + j * BUF, BUF)])
    plsc.subcore_barrier()

    for ph in range(NCHUNK // PCH):
        base_row = wid * NCHUNK + ph * PCH
        pltpu.sync_copy(src2_hbm.at[pl.ds(base_row, PCH)], sidx_v)
        pltpu.sync_copy(dst2_hbm.at[pl.ds(base_row, PCH)], didx_v)
        pltpu.async_copy(h_hbm.at[sidx_v.at[0]], buf0.at[pl.ds(0, CH)], gsem0)

        def step(g, carry):
            for b in range(2):
                i = g * 2 + b
                nb = 1 - b

                @pl.when(i + 1 < PCH)
                def _():
                    pltpu.async_copy(h_hbm.at[sidx_v.at[i + 1]],
                                     bufs[nb].at[pl.ds(0, CH)], gsems[nb])

                pltpu.make_async_copy(h_hbm.at[sidx_v.at[i]],
                                      bufs[b].at[pl.ds(0, CH)], gsems[b]).wait()
                pltpu.sync_copy(bufs[b].at[pl.ds(0, CH)],
                                acc_s.at[didx_v.at[i]], add=True)
            return carry

        lax.fori_loop(0, PCH // 2, step, 0)
    plsc.subcore_barrier()

    pltpu.sync_copy(acc_s.at[pl.ds(r0, RPT)], acc_out.at[c, pl.ds(r0, RPT)])


_sc_agg = pl.kernel(
    _sc_agg_body,
    out_type=[jax.ShapeDtypeStruct((NC, NPAD, D), jnp.float32)],
    mesh=_MESH,
    scratch_types=[
        pltpu.VMEM((PCH, CH), jnp.int32),
        pltpu.VMEM((PCH, CH), jnp.int32),
        pltpu.VMEM((BUF, D), jnp.float32),
        pltpu.VMEM((BUF, D), jnp.float32),
        pltpu.VMEM_SHARED((NPAD, D), jnp.float32),
        pltpu.SemaphoreType.DMA,
        pltpu.SemaphoreType.DMA,
    ],
)


def _sc_agg_deg_body(src2_hbm, dst2_hbm, h_hbm, acc_out, deg_out, sidx_v,
                     didx_v, buf0, buf1, acc_s, gsem0, gsem1, dsem):
    c = lax.axis_index("c")
    s = lax.axis_index("s")
    wid = s * NC + c
    r0 = s * RPT
    bufs = (buf0, buf1)
    gsems = (gsem0, gsem1)

    _zero_vmem_rows(buf0, BUF, D)
    for j in range(RPT // BUF):
        pltpu.sync_copy(buf0, acc_s.at[pl.ds(r0 + j * BUF, BUF)])
    plsc.subcore_barrier()

    for ph in range(NCHUNK // PCH):
        base_row = wid * NCHUNK + ph * PCH
        pltpu.sync_copy(src2_hbm.at[pl.ds(base_row, PCH)], sidx_v)
        pltpu.sync_copy(dst2_hbm.at[pl.ds(base_row, PCH)], didx_v)
        pltpu.async_copy(h_hbm.at[sidx_v.at[0]], buf0.at[pl.ds(0, CH)], gsem0)

        def step(g, carry):
            for b in range(2):
                i = g * 2 + b
                nb = 1 - b

                @pl.when(i + 1 < PCH)
                def _():
                    pltpu.async_copy(h_hbm.at[sidx_v.at[i + 1]],
                                     bufs[nb].at[pl.ds(0, CH)], gsems[nb])

                pltpu.make_async_copy(h_hbm.at[sidx_v.at[i]],
                                      bufs[b].at[pl.ds(0, CH)], gsems[b]).wait()
                pltpu.sync_copy(bufs[b].at[pl.ds(0, CH)],
                                acc_s.at[didx_v.at[i]], add=True)
            return carry

        lax.fori_loop(0, PCH // 2, step, 0)
    plsc.subcore_barrier()

    pltpu.sync_copy(acc_s.at[pl.ds(r0, RPT)], acc_out.at[c, pl.ds(r0, RPT)])

    _zero_vmem_rows(buf0, BUF, D)
    for j in range(RPT // BUF):
        pltpu.sync_copy(buf0, acc_s.at[pl.ds(r0 + j * BUF, BUF)])
    onevec = jnp.full((16,), 1.0, jnp.float32)

    def orow(i, carry):
        for j in range(D // 16):
            buf0[i, pl.ds(j * 16, 16)] = onevec
        return carry

    lax.fori_loop(0, BUF, orow, 0)
    plsc.subcore_barrier()

    DK = 8
    for ph in range(NCHUNK // PCH):
        pltpu.sync_copy(dst2_hbm.at[pl.ds(wid * NCHUNK + ph * PCH, PCH)],
                        didx_v)

        def dbody(gg, carry):
            for k in range(DK):
                pltpu.async_copy(buf0.at[pl.ds(0, CH)],
                                 acc_s.at[didx_v.at[gg * DK + k]], dsem,
                                 add=True)
            for k in range(DK):
                pltpu.make_async_copy(buf0.at[pl.ds(0, CH)],
                                      acc_s.at[didx_v.at[gg * DK + k]],
                                      dsem).wait()
            return carry

        lax.fori_loop(0, PCH // DK, dbody, 0)
    plsc.subcore_barrier()

    pltpu.sync_copy(acc_s.at[pl.ds(r0, RPT)], deg_out.at[c, pl.ds(r0, RPT)])


_sc_agg_deg = pl.kernel(
    _sc_agg_deg_body,
    out_type=[
        jax.ShapeDtypeStruct((NC, NPAD, D), jnp.float32),
        jax.ShapeDtypeStruct((NC, NPAD, D), jnp.float32),
    ],
    mesh=_MESH,
    scratch_types=[
        pltpu.VMEM((PCH, CH), jnp.int32),
        pltpu.VMEM((PCH, CH), jnp.int32),
        pltpu.VMEM((BUF, D), jnp.float32),
        pltpu.VMEM((BUF, D), jnp.float32),
        pltpu.VMEM_SHARED((NPAD, D), jnp.float32),
        pltpu.SemaphoreType.DMA,
        pltpu.SemaphoreType.DMA,
        pltpu.SemaphoreType.DMA,
    ],
)


def _inv_body(deg_ref, out_ref):
    d = deg_ref[0, :, 0:1] + deg_ref[1, :, 0:1]
    out_ref[...] = 1.0 / jnp.maximum(d, 1.0)


_IRB = 512


def _inv_deg(deg):
    return pl.pallas_call(
        _inv_body,
        grid=(NPAD // _IRB,),
        in_specs=[pl.BlockSpec((NC, _IRB, D), lambda i: (0, i, 0))],
        out_specs=pl.BlockSpec((_IRB, 1), lambda i: (i, 0)),
        out_shape=jax.ShapeDtypeStruct((NPAD, 1), jnp.float32),
    )(deg)


_RB = 400


def _layer_body(relu, fc, *refs):
    if fc:
        acc, inv, h, Wl, bl, Wr, Wfc, bfc, out = refs
    else:
        acc, inv, h, Wl, bl, Wr, out = refs
    agg = (acc[0] + acc[1]) * inv[...]
    y = (jnp.dot(agg, Wl[...], preferred_element_type=jnp.float32)
         + jnp.dot(h[...], Wr[...], preferred_element_type=jnp.float32)
         + bl[...])
    if relu:
        y = jnp.maximum(y, 0.0)
    if fc:
        y = jnp.dot(y, Wfc[...], preferred_element_type=jnp.float32) + bfc[...]
    out[...] = y


def _tc_layer(acc, inv, h, Wl, bl, Wr, Wfc=None, bfc=None, relu=True):
    fc = Wfc is not None
    in_specs = [
        pl.BlockSpec((NC, _RB, D), lambda i: (0, i, 0)),
        pl.BlockSpec((_RB, 1), lambda i: (i, 0)),
        pl.BlockSpec((_RB, D), lambda i: (i, 0)),
        pl.BlockSpec((D, D), lambda i: (0, 0)),
        pl.BlockSpec((1, D), lambda i: (0, 0)),
        pl.BlockSpec((D, D), lambda i: (0, 0)),
    ]
    args = [acc, inv, h, Wl, bl.reshape(1, D), Wr]
    if fc:
        in_specs += [
            pl.BlockSpec((D, D), lambda i: (0, 0)),
            pl.BlockSpec((1, D), lambda i: (0, 0)),
        ]
        args += [Wfc, bfc]
    return pl.pallas_call(
        functools.partial(_layer_body, relu, fc),
        grid=(N // _RB,),
        in_specs=in_specs,
        out_specs=pl.BlockSpec((_RB, D), lambda i: (i, 0)),
        out_shape=jax.ShapeDtypeStruct((N, D), jnp.float32),
    )(*args)


def kernel(x, edge_index, W1l, b1l, W1r, W2l, b2l, W2r, W3l, b3l, W3r, Wfc, bfc):
    src = edge_index[0].reshape(NW * NCHUNK, CH)
    dst = edge_index[1].reshape(NW * NCHUNK, CH)

    acc1, deg = _sc_agg_deg(src, dst, x)
    inv = _inv_deg(deg)
    h1 = _tc_layer(acc1, inv, x, W1l, b1l, W1r)
    (acc2,) = _sc_agg(src, dst, h1)
    h2 = _tc_layer(acc2, inv, h1, W2l, b2l, W2r)
    (acc3,) = _sc_agg(src, dst, h2)
    Wfc_p = jnp.zeros((D, D), jnp.float32).at[:, : Wfc.shape[1]].set(Wfc)
    bfc_p = jnp.zeros((1, D), jnp.float32).at[0, : bfc.shape[0]].set(bfc)
    pre = _tc_layer(acc3, inv, h2, W3l, b3l, W3r, Wfc_p, bfc_p, relu=False)
    return pre[:, : Wfc.shape[1]]
